# baseline (speedup 1.0000x reference)
import functools

import jax
import jax.numpy as jnp
from jax import lax
from jax.experimental import pallas as pl
from jax.experimental.pallas import tpu as pltpu
from jax.experimental.pallas import tpu_sc as plsc

_BLOCK_ROWS = 1024

_LOG1P_COEF = (
    1.472065e-06, 0.9998477, -0.49737322, 0.31574732, -0.19035433,
    0.08269124, -0.017414078,
)


def _log1p_poly(u):
    acc = jnp.full_like(u, _LOG1P_COEF[-1])
    for c in _LOG1P_COEF[-2::-1]:
        acc = acc * u + c
    return acc


def _bce(x, z):
    return jnp.maximum(x, 0.0) - x * z + jnp.log1p(jnp.exp(-jnp.abs(x)))


_SC_LANES = 16
_SC_CH_ROWS = 16
_SC_PW_ROWS = 32


def _sc_main_body(sc_row0, width, pred_hbm, gt_hbm, out_hbm,
                  x0_v, z0_v, x1_v, z1_v, acc_v,
                  sx0, sz0, sx1, sz1):
    nc = 2
    wid = lax.axis_index("s") * nc + lax.axis_index("c")
    n_chunks = _SC_PW_ROWS // _SC_CH_ROWS
    base_row = sc_row0 + wid * _SC_PW_ROWS

    bufs = ((x0_v, z0_v, sx0, sz0), (x1_v, z1_v, sx1, sz1))

    def start(c):
        xb, zb, sx, sz = bufs[c % 2]
        row = base_row + c * _SC_CH_ROWS
        hx = pltpu.async_copy(pred_hbm.at[pl.ds(row, _SC_CH_ROWS)], xb, sx)
        hz = pltpu.async_copy(gt_hbm.at[pl.ds(row, _SC_CH_ROWS)], zb, sz)
        return hx, hz

    def row_block(xb, zb, carry):
        def row_step(r, carry_r):
            def lane_step(j, carry2):
                ps, pc, ts = carry2
                x = xb[r, pl.ds(j * _SC_LANES, _SC_LANES)]
                z = zb[r, pl.ds(j * _SC_LANES, _SC_LANES)]
                xz = x * z
                s = x - xz - xz
                u = jnp.exp(-jnp.abs(s))
                loss = jnp.maximum(s, 0.0) + _log1p_poly(u)
                return ps + loss * z, pc + z, ts + loss

            return lax.fori_loop(0, width // _SC_LANES, lane_step, carry_r,
                                 unroll=8)

        return lax.fori_loop(0, _SC_CH_ROWS, row_step, carry)

    zero = jnp.zeros((_SC_LANES,), jnp.float32)
    carry = (zero, zero, zero)
    pend = start(0)
    for c in range(n_chunks):
        cur = pend
        if c + 1 < n_chunks:
            pend = start(c + 1)
        cur[0].wait()
        cur[1].wait()
        xb, zb, _, _ = bufs[c % 2]
        carry = row_block(xb, zb, carry)

    pos_s, pos_c, tot_s = carry
    acc_v[pl.ds(0, _SC_LANES)] = pos_s
    acc_v[pl.ds(_SC_LANES, _SC_LANES)] = pos_c
    acc_v[pl.ds(2 * _SC_LANES, _SC_LANES)] = tot_s
    pltpu.sync_copy(acc_v, out_hbm.at[pl.ds(wid * 3 * _SC_LANES, 3 * _SC_LANES)])


def _run_main_sc(p2, g2, sc_row0):
    width = p2.shape[1]
    mesh = plsc.VectorSubcoreMesh(core_axis_name="c", subcore_axis_name="s")
    f = functools.partial(
        pl.kernel,
        mesh=mesh,
        out_type=jax.ShapeDtypeStruct((32 * 3 * _SC_LANES,), jnp.float32),
        scratch_types=[
            pltpu.VMEM((_SC_CH_ROWS, width), jnp.float32),
            pltpu.VMEM((_SC_CH_ROWS, width), jnp.float32),
            pltpu.VMEM((_SC_CH_ROWS, width), jnp.float32),
            pltpu.VMEM((_SC_CH_ROWS, width), jnp.float32),
            pltpu.VMEM((3 * _SC_LANES,), jnp.float32),
            pltpu.SemaphoreType.DMA,
            pltpu.SemaphoreType.DMA,
            pltpu.SemaphoreType.DMA,
            pltpu.SemaphoreType.DMA,
        ],
    )(functools.partial(_sc_main_body, sc_row0, width))
    out = f(p2, g2)
    part = out.reshape(32, 3, _SC_LANES)
    sums = jnp.sum(part, axis=(0, 2))
    return sums


def _tc_main_body(pred_ref, gt_ref, out_ref):
    i = pl.program_id(0)
    x = pred_ref[...]
    z = gt_ref[...]
    loss = _bce(x, z)
    pos_sum = jnp.sum(loss * z)
    pos_cnt = jnp.sum(z)
    neg_sum = jnp.sum(loss * (1.0 - z))

    @pl.when(i == 0)
    def _():
        out_ref[0] = 0.0
        out_ref[1] = 0.0
        out_ref[2] = 0.0

    out_ref[0] += pos_sum
    out_ref[1] += pos_cnt
    out_ref[2] += neg_sum


_TC_BLOCK_ROWS = 512


def _run_main_tc(p2, g2, tc_rows):
    grid = tc_rows // _TC_BLOCK_ROWS
    return pl.pallas_call(
        _tc_main_body,
        grid=(grid,),
        in_specs=[
            pl.BlockSpec((_TC_BLOCK_ROWS, p2.shape[1]), lambda i: (i, 0)),
            pl.BlockSpec((_TC_BLOCK_ROWS, p2.shape[1]), lambda i: (i, 0)),
        ],
        out_specs=pl.BlockSpec(memory_space=pltpu.SMEM),
        out_shape=jax.ShapeDtypeStruct((3,), jnp.float32),
    )(p2, g2)


def _thresh_body(t_ref, pred_ref, gt_ref, out_ref):
    i = pl.program_id(0)
    t = t_ref[0]
    x = pred_ref[...]
    z = gt_ref[...]
    loss = _bce(x, z)
    vals = jnp.where(z == 0.0, loss, -1.0)
    cnt_ge = jnp.sum(jnp.where(vals >= t, 1.0, 0.0))
    cnt_gt = jnp.sum(jnp.where(vals > t, 1.0, 0.0))
    sum_gt = jnp.sum(jnp.where(vals > t, loss, 0.0))

    @pl.when(i == 0)
    def _():
        out_ref[0] = 0.0
        out_ref[1] = 0.0
        out_ref[2] = 0.0

    out_ref[0] += cnt_ge
    out_ref[1] += cnt_gt
    out_ref[2] += sum_gt


def _run_thresh(p2, g2, t):
    rows = p2.shape[0]
    grid = rows // _BLOCK_ROWS
    return pl.pallas_call(
        _thresh_body,
        grid=(grid,),
        in_specs=[
            pl.BlockSpec(memory_space=pltpu.SMEM),
            pl.BlockSpec((_BLOCK_ROWS, p2.shape[1]), lambda i: (i, 0)),
            pl.BlockSpec((_BLOCK_ROWS, p2.shape[1]), lambda i: (i, 0)),
        ],
        out_specs=pl.BlockSpec(memory_space=pltpu.SMEM),
        out_shape=jax.ShapeDtypeStruct((3,), jnp.float32),
    )(t.reshape(1), p2, g2)


def kernel(pred, gt):
    N, H, W = gt.shape
    total = float(N * H * W)
    eps = 1e-06
    p2 = pred.reshape(N * H, W)
    g2 = gt.reshape(N * H, W)

    sc_rows = 32 * _SC_PW_ROWS
    tc_rows = N * H - sc_rows

    tc_sums = _run_main_tc(p2, g2, tc_rows)
    sc_sums = _run_main_sc(p2, g2, tc_rows)
    pos_sum = tc_sums[0] + sc_sums[0]
    pos_cnt = tc_sums[1] + sc_sums[1]
    neg_sum_all = tc_sums[2] + (sc_sums[2] - sc_sums[0])
    neg_cnt_all = total - pos_cnt
    k = jnp.minimum(neg_cnt_all, 3.0 * pos_cnt)

    def fast_fn(_):
        return neg_sum_all

    def rare_fn(_):
        def body(_, carry):
            lo, hi = carry
            mid = lo + (hi - lo) // 2
            t = lax.bitcast_convert_type(mid, jnp.float32)
            c = _run_thresh(p2, g2, t)[0]
            take = c >= k
            return jnp.where(take, mid, lo), jnp.where(take, hi, mid)

        lo0 = jnp.int32(0)
        hi0 = jnp.int32(0x7F800000)
        lo, _hi = lax.fori_loop(0, 31, body, (lo0, hi0))
        t = lax.bitcast_convert_type(lo, jnp.float32)
        o = _run_thresh(p2, g2, t)
        cnt_gt, sum_gt = o[1], o[2]
        return jnp.where(k > 0.0, sum_gt + (k - cnt_gt) * t, 0.0)

    neg_sum = lax.cond(k >= neg_cnt_all, fast_fn, rare_fn, None)
    return (pos_sum + neg_sum) / (pos_cnt + k + eps)


# speedup vs baseline: 1.0394x; 1.0394x over previous
import functools

import jax
import jax.numpy as jnp
from jax import lax
from jax.experimental import pallas as pl
from jax.experimental.pallas import tpu as pltpu
from jax.experimental.pallas import tpu_sc as plsc

_BLOCK_ROWS = 1024

_LOG1P_COEF = (
    1.472065e-06, 0.9998477, -0.49737322, 0.31574732, -0.19035433,
    0.08269124, -0.017414078,
)


def _log1p_poly(u):
    acc = jnp.full_like(u, _LOG1P_COEF[-1])
    for c in _LOG1P_COEF[-2::-1]:
        acc = acc * u + c
    return acc


def _bce(x, z):
    return jnp.maximum(x, 0.0) - x * z + jnp.log1p(jnp.exp(-jnp.abs(x)))


_SC_LANES = 16
_SC_CH_ROWS = 16
_SC_PW_ROWS = 48


def _sc_main_body(sc_row0, width, pred_hbm, gt_hbm, out_hbm,
                  x0_v, z0_v, x1_v, z1_v, acc_v,
                  sx0, sz0, sx1, sz1):
    nc = 2
    wid = lax.axis_index("s") * nc + lax.axis_index("c")
    n_chunks = _SC_PW_ROWS // _SC_CH_ROWS
    base_row = sc_row0 + wid * _SC_PW_ROWS

    bufs = ((x0_v, z0_v, sx0, sz0), (x1_v, z1_v, sx1, sz1))

    def start(c):
        xb, zb, sx, sz = bufs[c % 2]
        row = base_row + c * _SC_CH_ROWS
        hx = pltpu.async_copy(pred_hbm.at[pl.ds(row, _SC_CH_ROWS)], xb, sx)
        hz = pltpu.async_copy(gt_hbm.at[pl.ds(row, _SC_CH_ROWS)], zb, sz)
        return hx, hz

    def row_block(xb, zb, carry):
        def row_step(r, carry_r):
            def lane_step(j, carry2):
                ps, pc, ts = carry2
                x = xb[r, pl.ds(j * _SC_LANES, _SC_LANES)]
                z = zb[r, pl.ds(j * _SC_LANES, _SC_LANES)]
                xz = x * z
                s = x - xz - xz
                u = jnp.exp(-jnp.abs(s))
                loss = jnp.maximum(s, 0.0) + _log1p_poly(u)
                return ps + loss * z, pc + z, ts + loss

            return lax.fori_loop(0, width // _SC_LANES, lane_step, carry_r,
                                 unroll=8)

        return lax.fori_loop(0, _SC_CH_ROWS, row_step, carry)

    zero = jnp.zeros((_SC_LANES,), jnp.float32)
    carry = (zero, zero, zero)
    pend = start(0)
    for c in range(n_chunks):
        cur = pend
        if c + 1 < n_chunks:
            pend = start(c + 1)
        cur[0].wait()
        cur[1].wait()
        xb, zb, _, _ = bufs[c % 2]
        carry = row_block(xb, zb, carry)

    pos_s, pos_c, tot_s = carry
    acc_v[pl.ds(0, _SC_LANES)] = pos_s
    acc_v[pl.ds(_SC_LANES, _SC_LANES)] = pos_c
    acc_v[pl.ds(2 * _SC_LANES, _SC_LANES)] = tot_s
    pltpu.sync_copy(acc_v, out_hbm.at[wid])


def _run_main_sc(p2, g2, sc_row0):
    width = p2.shape[1]
    mesh = plsc.VectorSubcoreMesh(core_axis_name="c", subcore_axis_name="s")
    f = functools.partial(
        pl.kernel,
        mesh=mesh,
        out_type=jax.ShapeDtypeStruct((32, 3 * _SC_LANES), jnp.float32),
        scratch_types=[
            pltpu.VMEM((_SC_CH_ROWS, width), jnp.float32),
            pltpu.VMEM((_SC_CH_ROWS, width), jnp.float32),
            pltpu.VMEM((_SC_CH_ROWS, width), jnp.float32),
            pltpu.VMEM((_SC_CH_ROWS, width), jnp.float32),
            pltpu.VMEM((3 * _SC_LANES,), jnp.float32),
            pltpu.SemaphoreType.DMA,
            pltpu.SemaphoreType.DMA,
            pltpu.SemaphoreType.DMA,
            pltpu.SemaphoreType.DMA,
        ],
    )(functools.partial(_sc_main_body, sc_row0, width))
    out = f(p2, g2)
    part = out.reshape(32, 3, _SC_LANES)
    sums = jnp.sum(part, axis=(0, 2))
    return sums


def _tc_main_body(pred_ref, gt_ref, out_ref):
    i = pl.program_id(0)
    x = pred_ref[...]
    z = gt_ref[...]
    loss = _bce(x, z)
    pos_sum = jnp.sum(loss * z)
    pos_cnt = jnp.sum(z)
    neg_sum = jnp.sum(loss * (1.0 - z))

    @pl.when(i == 0)
    def _():
        out_ref[0] = 0.0
        out_ref[1] = 0.0
        out_ref[2] = 0.0

    out_ref[0] += pos_sum
    out_ref[1] += pos_cnt
    out_ref[2] += neg_sum


_TC_BLOCK_ROWS = 512


def _run_main_tc(p2, g2, tc_rows):
    grid = tc_rows // _TC_BLOCK_ROWS
    return pl.pallas_call(
        _tc_main_body,
        grid=(grid,),
        in_specs=[
            pl.BlockSpec((_TC_BLOCK_ROWS, p2.shape[1]), lambda i: (i, 0)),
            pl.BlockSpec((_TC_BLOCK_ROWS, p2.shape[1]), lambda i: (i, 0)),
        ],
        out_specs=pl.BlockSpec(memory_space=pltpu.SMEM),
        out_shape=jax.ShapeDtypeStruct((3,), jnp.float32),
    )(p2, g2)


def _thresh_body(t_ref, pred_ref, gt_ref, out_ref):
    i = pl.program_id(0)
    t = t_ref[0]
    x = pred_ref[...]
    z = gt_ref[...]
    loss = _bce(x, z)
    vals = jnp.where(z == 0.0, loss, -1.0)
    cnt_ge = jnp.sum(jnp.where(vals >= t, 1.0, 0.0))
    cnt_gt = jnp.sum(jnp.where(vals > t, 1.0, 0.0))
    sum_gt = jnp.sum(jnp.where(vals > t, loss, 0.0))

    @pl.when(i == 0)
    def _():
        out_ref[0] = 0.0
        out_ref[1] = 0.0
        out_ref[2] = 0.0

    out_ref[0] += cnt_ge
    out_ref[1] += cnt_gt
    out_ref[2] += sum_gt


def _run_thresh(p2, g2, t):
    rows = p2.shape[0]
    grid = rows // _BLOCK_ROWS
    return pl.pallas_call(
        _thresh_body,
        grid=(grid,),
        in_specs=[
            pl.BlockSpec(memory_space=pltpu.SMEM),
            pl.BlockSpec((_BLOCK_ROWS, p2.shape[1]), lambda i: (i, 0)),
            pl.BlockSpec((_BLOCK_ROWS, p2.shape[1]), lambda i: (i, 0)),
        ],
        out_specs=pl.BlockSpec(memory_space=pltpu.SMEM),
        out_shape=jax.ShapeDtypeStruct((3,), jnp.float32),
    )(t.reshape(1), p2, g2)


def kernel(pred, gt):
    N, H, W = gt.shape
    total = float(N * H * W)
    eps = 1e-06
    p2 = pred.reshape(N * H, W)
    g2 = gt.reshape(N * H, W)

    sc_rows = 32 * _SC_PW_ROWS
    tc_rows = N * H - sc_rows

    tc_sums = _run_main_tc(p2, g2, tc_rows)
    sc_sums = _run_main_sc(p2, g2, tc_rows)
    pos_sum = tc_sums[0] + sc_sums[0]
    pos_cnt = tc_sums[1] + sc_sums[1]
    neg_sum_all = tc_sums[2] + (sc_sums[2] - sc_sums[0])
    neg_cnt_all = total - pos_cnt
    k = jnp.minimum(neg_cnt_all, 3.0 * pos_cnt)

    def fast_fn(_):
        return neg_sum_all

    def rare_fn(_):
        def body(_, carry):
            lo, hi = carry
            mid = lo + (hi - lo) // 2
            t = lax.bitcast_convert_type(mid, jnp.float32)
            c = _run_thresh(p2, g2, t)[0]
            take = c >= k
            return jnp.where(take, mid, lo), jnp.where(take, hi, mid)

        lo0 = jnp.int32(0)
        hi0 = jnp.int32(0x7F800000)
        lo, _hi = lax.fori_loop(0, 31, body, (lo0, hi0))
        t = lax.bitcast_convert_type(lo, jnp.float32)
        o = _run_thresh(p2, g2, t)
        cnt_gt, sum_gt = o[1], o[2]
        return jnp.where(k > 0.0, sum_gt + (k - cnt_gt) * t, 0.0)

    neg_sum = lax.cond(k >= neg_cnt_all, fast_fn, rare_fn, None)
    return (pos_sum + neg_sum) / (pos_cnt + k + eps)


# speedup vs baseline: 1.0632x; 1.0228x over previous
import functools

import jax
import jax.numpy as jnp
from jax import lax
from jax.experimental import pallas as pl
from jax.experimental.pallas import tpu as pltpu
from jax.experimental.pallas import tpu_sc as plsc

_BLOCK_ROWS = 1024

_LOG1P_COEF = (
    1.472065e-06, 0.9998477, -0.49737322, 0.31574732, -0.19035433,
    0.08269124, -0.017414078,
)


def _log1p_poly(u):
    acc = jnp.full_like(u, _LOG1P_COEF[-1])
    for c in _LOG1P_COEF[-2::-1]:
        acc = acc * u + c
    return acc


def _bce(x, z):
    return jnp.maximum(x, 0.0) - x * z + jnp.log1p(jnp.exp(-jnp.abs(x)))


_SC_LANES = 16
_SC_CH_ROWS = 16
_SC_PW_ROWS = 48


def _sc_main_body(sc_row0, width, pred_hbm, gt_hbm, out_hbm,
                  x0_v, z0_v, x1_v, z1_v, acc_v,
                  sx0, sz0, sx1, sz1):
    nc = 2
    wid = lax.axis_index("s") * nc + lax.axis_index("c")
    n_chunks = _SC_PW_ROWS // _SC_CH_ROWS
    base_row = sc_row0 + wid * _SC_PW_ROWS

    bufs = ((x0_v, z0_v, sx0, sz0), (x1_v, z1_v, sx1, sz1))

    def start(c):
        xb, zb, sx, sz = bufs[c % 2]
        row = base_row + c * _SC_CH_ROWS
        hx = pltpu.async_copy(pred_hbm.at[pl.ds(row, _SC_CH_ROWS)], xb, sx)
        hz = pltpu.async_copy(gt_hbm.at[pl.ds(row, _SC_CH_ROWS)], zb, sz)
        return hx, hz

    def row_block(xb, zb, carry):
        def row_step(r, carry_r):
            def lane_step(j, carry2):
                ps, pc, ts = carry2
                x = xb[r, pl.ds(j * _SC_LANES, _SC_LANES)]
                z = zb[r, pl.ds(j * _SC_LANES, _SC_LANES)]
                xz = x * z
                s = x - xz - xz
                u = jnp.exp(-jnp.abs(s))
                loss = jnp.maximum(s, 0.0) + _log1p_poly(u)
                return ps + loss * z, pc + z, ts + loss

            return lax.fori_loop(0, width // _SC_LANES, lane_step, carry_r,
                                 unroll=8)

        return lax.fori_loop(0, _SC_CH_ROWS, row_step, carry)

    zero = jnp.zeros((_SC_LANES,), jnp.float32)
    carry = (zero, zero, zero)
    pend = start(0)
    for c in range(n_chunks):
        cur = pend
        if c + 1 < n_chunks:
            pend = start(c + 1)
        cur[0].wait()
        cur[1].wait()
        xb, zb, _, _ = bufs[c % 2]
        carry = row_block(xb, zb, carry)

    pos_s, pos_c, tot_s = carry
    acc_v[pl.ds(0, _SC_LANES)] = pos_s
    acc_v[pl.ds(_SC_LANES, _SC_LANES)] = pos_c
    acc_v[pl.ds(2 * _SC_LANES, _SC_LANES)] = tot_s
    pltpu.sync_copy(acc_v, out_hbm.at[wid])


def _run_main_sc(p2, g2, sc_row0):
    width = p2.shape[1]
    mesh = plsc.VectorSubcoreMesh(core_axis_name="c", subcore_axis_name="s")
    f = functools.partial(
        pl.kernel,
        mesh=mesh,
        out_type=jax.ShapeDtypeStruct((32, 3 * _SC_LANES), jnp.float32),
        scratch_types=[
            pltpu.VMEM((_SC_CH_ROWS, width), jnp.float32),
            pltpu.VMEM((_SC_CH_ROWS, width), jnp.float32),
            pltpu.VMEM((_SC_CH_ROWS, width), jnp.float32),
            pltpu.VMEM((_SC_CH_ROWS, width), jnp.float32),
            pltpu.VMEM((3 * _SC_LANES,), jnp.float32),
            pltpu.SemaphoreType.DMA,
            pltpu.SemaphoreType.DMA,
            pltpu.SemaphoreType.DMA,
            pltpu.SemaphoreType.DMA,
        ],
    )(functools.partial(_sc_main_body, sc_row0, width))
    out = f(p2, g2)
    part = out.reshape(32, 3, _SC_LANES)
    sums = jnp.sum(part, axis=(0, 2))
    return sums


def _tc_main_body(pred_ref, gt_ref, out_ref):
    i = pl.program_id(0)
    x = pred_ref[...]
    z = gt_ref[...]
    loss = _bce(x, z)
    pos_sum = jnp.sum(loss * z)
    pos_cnt = jnp.sum(z)
    tot_sum = jnp.sum(loss)

    @pl.when(i == 0)
    def _():
        out_ref[0] = 0.0
        out_ref[1] = 0.0
        out_ref[2] = 0.0

    out_ref[0] += pos_sum
    out_ref[1] += pos_cnt
    out_ref[2] += tot_sum


_TC_BLOCK_ROWS = 512


def _run_main_tc(p2, g2, tc_rows):
    grid = tc_rows // _TC_BLOCK_ROWS
    return pl.pallas_call(
        _tc_main_body,
        grid=(grid,),
        in_specs=[
            pl.BlockSpec((_TC_BLOCK_ROWS, p2.shape[1]), lambda i: (i, 0)),
            pl.BlockSpec((_TC_BLOCK_ROWS, p2.shape[1]), lambda i: (i, 0)),
        ],
        out_specs=pl.BlockSpec(memory_space=pltpu.SMEM),
        out_shape=jax.ShapeDtypeStruct((3,), jnp.float32),
    )(p2, g2)


def _thresh_body(t_ref, pred_ref, gt_ref, out_ref):
    i = pl.program_id(0)
    t = t_ref[0]
    x = pred_ref[...]
    z = gt_ref[...]
    loss = _bce(x, z)
    vals = jnp.where(z == 0.0, loss, -1.0)
    cnt_ge = jnp.sum(jnp.where(vals >= t, 1.0, 0.0))
    cnt_gt = jnp.sum(jnp.where(vals > t, 1.0, 0.0))
    sum_gt = jnp.sum(jnp.where(vals > t, loss, 0.0))

    @pl.when(i == 0)
    def _():
        out_ref[0] = 0.0
        out_ref[1] = 0.0
        out_ref[2] = 0.0

    out_ref[0] += cnt_ge
    out_ref[1] += cnt_gt
    out_ref[2] += sum_gt


def _run_thresh(p2, g2, t):
    rows = p2.shape[0]
    grid = rows // _BLOCK_ROWS
    return pl.pallas_call(
        _thresh_body,
        grid=(grid,),
        in_specs=[
            pl.BlockSpec(memory_space=pltpu.SMEM),
            pl.BlockSpec((_BLOCK_ROWS, p2.shape[1]), lambda i: (i, 0)),
            pl.BlockSpec((_BLOCK_ROWS, p2.shape[1]), lambda i: (i, 0)),
        ],
        out_specs=pl.BlockSpec(memory_space=pltpu.SMEM),
        out_shape=jax.ShapeDtypeStruct((3,), jnp.float32),
    )(t.reshape(1), p2, g2)


def kernel(pred, gt):
    N, H, W = gt.shape
    total = float(N * H * W)
    eps = 1e-06
    p2 = pred.reshape(N * H, W)
    g2 = gt.reshape(N * H, W)

    sc_rows = 32 * _SC_PW_ROWS
    tc_rows = N * H - sc_rows

    tc_sums = _run_main_tc(p2, g2, tc_rows)
    sc_sums = _run_main_sc(p2, g2, tc_rows)
    pos_sum = tc_sums[0] + sc_sums[0]
    pos_cnt = tc_sums[1] + sc_sums[1]
    neg_sum_all = (tc_sums[2] + sc_sums[2]) - pos_sum
    neg_cnt_all = total - pos_cnt
    k = jnp.minimum(neg_cnt_all, 3.0 * pos_cnt)

    def fast_fn(_):
        return neg_sum_all

    def rare_fn(_):
        def body(_, carry):
            lo, hi = carry
            mid = lo + (hi - lo) // 2
            t = lax.bitcast_convert_type(mid, jnp.float32)
            c = _run_thresh(p2, g2, t)[0]
            take = c >= k
            return jnp.where(take, mid, lo), jnp.where(take, hi, mid)

        lo0 = jnp.int32(0)
        hi0 = jnp.int32(0x7F800000)
        lo, _hi = lax.fori_loop(0, 31, body, (lo0, hi0))
        t = lax.bitcast_convert_type(lo, jnp.float32)
        o = _run_thresh(p2, g2, t)
        cnt_gt, sum_gt = o[1], o[2]
        return jnp.where(k > 0.0, sum_gt + (k - cnt_gt) * t, 0.0)

    neg_sum = lax.cond(k >= neg_cnt_all, fast_fn, rare_fn, None)
    return (pos_sum + neg_sum) / (pos_cnt + k + eps)


# speedup vs baseline: 1.0809x; 1.0167x over previous
import functools

import jax
import jax.numpy as jnp
from jax import lax
from jax.experimental import pallas as pl
from jax.experimental.pallas import tpu as pltpu
from jax.experimental.pallas import tpu_sc as plsc

_BLOCK_ROWS = 1024

_LOG1P_COEF = (
    1.472065e-06, 0.9998477, -0.49737322, 0.31574732, -0.19035433,
    0.08269124, -0.017414078,
)


def _log1p_poly(u):
    acc = jnp.full_like(u, _LOG1P_COEF[-1])
    for c in _LOG1P_COEF[-2::-1]:
        acc = acc * u + c
    return acc


def _bce(x, z):
    return jnp.maximum(x, 0.0) - x * z + jnp.log1p(jnp.exp(-jnp.abs(x)))


_SC_LANES = 16
_SC_CH_ROWS = 16
_SC_PW_ROWS = 64


def _sc_main_body(sc_row0, width, pred_hbm, gt_hbm, out_hbm,
                  x0_v, z0_v, x1_v, z1_v, acc_v,
                  sx0, sz0, sx1, sz1):
    nc = 2
    wid = lax.axis_index("s") * nc + lax.axis_index("c")
    n_chunks = _SC_PW_ROWS // _SC_CH_ROWS
    base_row = sc_row0 + wid * _SC_PW_ROWS

    bufs = ((x0_v, z0_v, sx0, sz0), (x1_v, z1_v, sx1, sz1))

    def start(c):
        xb, zb, sx, sz = bufs[c % 2]
        row = base_row + c * _SC_CH_ROWS
        hx = pltpu.async_copy(pred_hbm.at[pl.ds(row, _SC_CH_ROWS)], xb, sx)
        hz = pltpu.async_copy(gt_hbm.at[pl.ds(row, _SC_CH_ROWS)], zb, sz)
        return hx, hz

    def row_block(xb, zb, carry):
        def row_step(r, carry_r):
            def lane_step(j, carry2):
                ps, pc, ts = carry2
                x = xb[r, pl.ds(j * _SC_LANES, _SC_LANES)]
                z = zb[r, pl.ds(j * _SC_LANES, _SC_LANES)]
                xz = x * z
                s = x - xz - xz
                u = jnp.exp(-jnp.abs(s))
                loss = jnp.maximum(s, 0.0) + _log1p_poly(u)
                return ps + loss * z, pc + z, ts + loss

            return lax.fori_loop(0, width // _SC_LANES, lane_step, carry_r,
                                 unroll=8)

        return lax.fori_loop(0, _SC_CH_ROWS, row_step, carry)

    zero = jnp.zeros((_SC_LANES,), jnp.float32)
    carry = (zero, zero, zero)
    pend = start(0)
    for c in range(n_chunks):
        cur = pend
        if c + 1 < n_chunks:
            pend = start(c + 1)
        cur[0].wait()
        cur[1].wait()
        xb, zb, _, _ = bufs[c % 2]
        carry = row_block(xb, zb, carry)

    pos_s, pos_c, tot_s = carry
    acc_v[pl.ds(0, _SC_LANES)] = pos_s
    acc_v[pl.ds(_SC_LANES, _SC_LANES)] = pos_c
    acc_v[pl.ds(2 * _SC_LANES, _SC_LANES)] = tot_s
    pltpu.sync_copy(acc_v, out_hbm.at[wid])


def _run_main_sc(p2, g2, sc_row0):
    width = p2.shape[1]
    mesh = plsc.VectorSubcoreMesh(core_axis_name="c", subcore_axis_name="s")
    f = functools.partial(
        pl.kernel,
        mesh=mesh,
        out_type=jax.ShapeDtypeStruct((32, 3 * _SC_LANES), jnp.float32),
        scratch_types=[
            pltpu.VMEM((_SC_CH_ROWS, width), jnp.float32),
            pltpu.VMEM((_SC_CH_ROWS, width), jnp.float32),
            pltpu.VMEM((_SC_CH_ROWS, width), jnp.float32),
            pltpu.VMEM((_SC_CH_ROWS, width), jnp.float32),
            pltpu.VMEM((3 * _SC_LANES,), jnp.float32),
            pltpu.SemaphoreType.DMA,
            pltpu.SemaphoreType.DMA,
            pltpu.SemaphoreType.DMA,
            pltpu.SemaphoreType.DMA,
        ],
    )(functools.partial(_sc_main_body, sc_row0, width))
    out = f(p2, g2)
    part = out.reshape(32, 3, _SC_LANES)
    sums = jnp.sum(part, axis=(0, 2))
    return sums


def _tc_main_body(pred_ref, gt_ref, out_ref):
    i = pl.program_id(0)
    x = pred_ref[...]
    z = gt_ref[...]
    loss = _bce(x, z)
    pos_sum = jnp.sum(loss * z)
    pos_cnt = jnp.sum(z)
    tot_sum = jnp.sum(loss)

    @pl.when(i == 0)
    def _():
        out_ref[0] = 0.0
        out_ref[1] = 0.0
        out_ref[2] = 0.0

    out_ref[0] += pos_sum
    out_ref[1] += pos_cnt
    out_ref[2] += tot_sum


_TC_BLOCK_ROWS = 512


def _run_main_tc(p2, g2, tc_rows):
    grid = tc_rows // _TC_BLOCK_ROWS
    return pl.pallas_call(
        _tc_main_body,
        grid=(grid,),
        in_specs=[
            pl.BlockSpec((_TC_BLOCK_ROWS, p2.shape[1]), lambda i: (i, 0)),
            pl.BlockSpec((_TC_BLOCK_ROWS, p2.shape[1]), lambda i: (i, 0)),
        ],
        out_specs=pl.BlockSpec(memory_space=pltpu.SMEM),
        out_shape=jax.ShapeDtypeStruct((3,), jnp.float32),
    )(p2, g2)


def _thresh_body(t_ref, pred_ref, gt_ref, out_ref):
    i = pl.program_id(0)
    t = t_ref[0]
    x = pred_ref[...]
    z = gt_ref[...]
    loss = _bce(x, z)
    vals = jnp.where(z == 0.0, loss, -1.0)
    cnt_ge = jnp.sum(jnp.where(vals >= t, 1.0, 0.0))
    cnt_gt = jnp.sum(jnp.where(vals > t, 1.0, 0.0))
    sum_gt = jnp.sum(jnp.where(vals > t, loss, 0.0))

    @pl.when(i == 0)
    def _():
        out_ref[0] = 0.0
        out_ref[1] = 0.0
        out_ref[2] = 0.0

    out_ref[0] += cnt_ge
    out_ref[1] += cnt_gt
    out_ref[2] += sum_gt


def _run_thresh(p2, g2, t):
    rows = p2.shape[0]
    grid = rows // _BLOCK_ROWS
    return pl.pallas_call(
        _thresh_body,
        grid=(grid,),
        in_specs=[
            pl.BlockSpec(memory_space=pltpu.SMEM),
            pl.BlockSpec((_BLOCK_ROWS, p2.shape[1]), lambda i: (i, 0)),
            pl.BlockSpec((_BLOCK_ROWS, p2.shape[1]), lambda i: (i, 0)),
        ],
        out_specs=pl.BlockSpec(memory_space=pltpu.SMEM),
        out_shape=jax.ShapeDtypeStruct((3,), jnp.float32),
    )(t.reshape(1), p2, g2)


def kernel(pred, gt):
    N, H, W = gt.shape
    total = float(N * H * W)
    eps = 1e-06
    p2 = pred.reshape(N * H, W)
    g2 = gt.reshape(N * H, W)

    sc_rows = 32 * _SC_PW_ROWS
    tc_rows = N * H - sc_rows

    tc_sums = _run_main_tc(p2, g2, tc_rows)
    sc_sums = _run_main_sc(p2, g2, tc_rows)
    pos_sum = tc_sums[0] + sc_sums[0]
    pos_cnt = tc_sums[1] + sc_sums[1]
    neg_sum_all = (tc_sums[2] + sc_sums[2]) - pos_sum
    neg_cnt_all = total - pos_cnt
    k = jnp.minimum(neg_cnt_all, 3.0 * pos_cnt)

    def fast_fn(_):
        return neg_sum_all

    def rare_fn(_):
        def body(_, carry):
            lo, hi = carry
            mid = lo + (hi - lo) // 2
            t = lax.bitcast_convert_type(mid, jnp.float32)
            c = _run_thresh(p2, g2, t)[0]
            take = c >= k
            return jnp.where(take, mid, lo), jnp.where(take, hi, mid)

        lo0 = jnp.int32(0)
        hi0 = jnp.int32(0x7F800000)
        lo, _hi = lax.fori_loop(0, 31, body, (lo0, hi0))
        t = lax.bitcast_convert_type(lo, jnp.float32)
        o = _run_thresh(p2, g2, t)
        cnt_gt, sum_gt = o[1], o[2]
        return jnp.where(k > 0.0, sum_gt + (k - cnt_gt) * t, 0.0)

    neg_sum = lax.cond(k >= neg_cnt_all, fast_fn, rare_fn, None)
    return (pos_sum + neg_sum) / (pos_cnt + k + eps)
